# baseline (speedup 1.0000x reference)
import math

import jax
import jax.numpy as jnp
from jax.experimental import pallas as pl
from jax.experimental.pallas import tpu as pltpu

_N = 1024
_D = 64
_H = 2
_NP = 2
_PW = _N // _NP


def _attn_body(x_ref, sign_ref, adj0_ref, adj1_ref,
               wq_ref, bq_ref, wk_ref, bk_ref, wv_ref, bv_ref,
               wo_ref, bo_ref, out_ref):
    inv_sqrt_d = 1.0 / math.sqrt(_D)
    bf16 = jnp.bfloat16

    x = x_ref[:].astype(bf16)
    k_all = (jnp.dot(x, wk_ref[:].T.astype(bf16),
                     preferred_element_type=jnp.float32) + bk_ref[:]).astype(bf16)
    v_all = (jnp.dot(x, wv_ref[:].T.astype(bf16),
                     preferred_element_type=jnp.float32) + bv_ref[:]).astype(bf16)
    q_blk = jnp.dot(x, wq_ref[:].T.astype(bf16),
                    preferred_element_type=jnp.float32) + bq_ref[:]
    q_blk = (q_blk * (sign_ref[:] * inv_sqrt_d)).astype(bf16)

    adj_refs = (adj0_ref, adj1_ref)

    heads = []
    for h in range(_H):
        qh = q_blk[:, h * _D:(h + 1) * _D]
        kh = k_all[:, h * _D:(h + 1) * _D]
        vh = v_all[:, h * _D:(h + 1) * _D]
        s = jnp.dot(qh, kh.T, preferred_element_type=jnp.float32)
        m = jnp.max(s, axis=1, keepdims=True)
        wv = jnp.zeros((_N, _D), jnp.float32)
        denom = jnp.float32(1e-10)
        for p in range(_NP):
            mask = adj_refs[p][:] > 0.0
            sp = s[:, p * _PW:(p + 1) * _PW]
            w = jnp.where(mask, jnp.exp(sp - m), 0.0).astype(bf16)
            denom = denom + jnp.sum(w.astype(jnp.float32), axis=1, keepdims=True)
            wv = wv + jnp.dot(w, vh[p * _PW:(p + 1) * _PW, :],
                              preferred_element_type=jnp.float32)
        heads.append(wv / denom)

    out_heads = jnp.concatenate(heads, axis=1).astype(bf16)
    out_ref[:] = (jnp.dot(out_heads, wo_ref[:].T.astype(bf16),
                          preferred_element_type=jnp.float32)
                  + bo_ref[:])


def kernel(node_embeddings, node_sign_influence, adj_matrix,
           Wq, bq, Wk, bk, Wv, bv, Wo, bo, sign_weight):
    del sign_weight
    n = node_embeddings.shape[0]
    sign2d = node_sign_influence.reshape(n, 1)
    return pl.pallas_call(
        _attn_body,
        grid=(1,),
        in_specs=[
            pl.BlockSpec((n, _D), lambda i: (0, 0)),
            pl.BlockSpec((n, 1), lambda i: (0, 0)),
            pl.BlockSpec((n, _PW), lambda i: (0, 0)),
            pl.BlockSpec((n, _PW), lambda i: (0, 1)),
            pl.BlockSpec((_D * _H, _D), lambda i: (0, 0)),
            pl.BlockSpec((1, _D * _H), lambda i: (0, 0)),
            pl.BlockSpec((_D * _H, _D), lambda i: (0, 0)),
            pl.BlockSpec((1, _D * _H), lambda i: (0, 0)),
            pl.BlockSpec((_D * _H, _D), lambda i: (0, 0)),
            pl.BlockSpec((1, _D * _H), lambda i: (0, 0)),
            pl.BlockSpec((_D, _D * _H), lambda i: (0, 0)),
            pl.BlockSpec((1, _D), lambda i: (0, 0)),
        ],
        out_specs=pl.BlockSpec((n, _D), lambda i: (0, 0)),
        out_shape=jax.ShapeDtypeStruct((n, _D), jnp.float32),
    )(node_embeddings, sign2d, adj_matrix, adj_matrix,
      Wq, bq.reshape(1, -1), Wk, bk.reshape(1, -1), Wv, bv.reshape(1, -1),
      Wo, bo.reshape(1, -1))


# speedup vs baseline: 1.1124x; 1.1124x over previous
import math

import jax
import jax.numpy as jnp
from jax.experimental import pallas as pl
from jax.experimental.pallas import tpu as pltpu

_N = 1024
_D = 64
_H = 2
_NP = 2
_PW = _N // _NP


def _attn_body(x_ref, sign_ref, adj0_ref, adj1_ref,
               wq_ref, bq_ref, wk_ref, bk_ref, wv_ref, bv_ref,
               wo_ref, bo_ref, out_ref):
    inv_sqrt_d = 1.0 / math.sqrt(_D)
    bf16 = jnp.bfloat16

    x = x_ref[:].astype(bf16)
    k_all = (jnp.dot(x, wk_ref[:].T.astype(bf16),
                     preferred_element_type=jnp.float32)
             + bk_ref[:][None, :]).astype(bf16)
    v_all = (jnp.dot(x, wv_ref[:].T.astype(bf16),
                     preferred_element_type=jnp.float32)
             + bv_ref[:][None, :]).astype(bf16)
    q_blk = (jnp.dot(x, wq_ref[:].T.astype(bf16),
                     preferred_element_type=jnp.float32)
             + bq_ref[:][None, :])
    sign_col = sign_ref[:].reshape(_N, 1)
    q_blk = (q_blk * (sign_col * inv_sqrt_d)).astype(bf16)

    adj_refs = (adj0_ref, adj1_ref)

    heads = []
    for h in range(_H):
        qh = q_blk[:, h * _D:(h + 1) * _D]
        kh = k_all[:, h * _D:(h + 1) * _D]
        vh = v_all[:, h * _D:(h + 1) * _D]
        s = jnp.dot(qh, kh.T, preferred_element_type=jnp.float32)
        m = jnp.max(s, axis=1, keepdims=True)
        wv = jnp.zeros((_N, _D), jnp.float32)
        denom = jnp.float32(1e-10)
        for p in range(_NP):
            mask = adj_refs[p][:] > 0.0
            sp = s[:, p * _PW:(p + 1) * _PW]
            w = jnp.where(mask, jnp.exp(sp - m), 0.0).astype(bf16)
            denom = denom + jnp.sum(w.astype(jnp.float32), axis=1, keepdims=True)
            wv = wv + jnp.dot(w, vh[p * _PW:(p + 1) * _PW, :],
                              preferred_element_type=jnp.float32)
        heads.append(wv / denom)

    out_heads = jnp.concatenate(heads, axis=1).astype(bf16)
    out_ref[:] = (jnp.dot(out_heads, wo_ref[:].T.astype(bf16),
                          preferred_element_type=jnp.float32)
                  + bo_ref[:][None, :])


def kernel(node_embeddings, node_sign_influence, adj_matrix,
           Wq, bq, Wk, bk, Wv, bv, Wo, bo, sign_weight):
    del sign_weight
    n = node_embeddings.shape[0]
    return pl.pallas_call(
        _attn_body,
        grid=(1,),
        in_specs=[
            pl.BlockSpec((n, _D), lambda i: (0, 0)),
            pl.BlockSpec((n,), lambda i: (0,)),
            pl.BlockSpec((n, _PW), lambda i: (0, 0)),
            pl.BlockSpec((n, _PW), lambda i: (0, 1)),
            pl.BlockSpec((_D * _H, _D), lambda i: (0, 0)),
            pl.BlockSpec((_D * _H,), lambda i: (0,)),
            pl.BlockSpec((_D * _H, _D), lambda i: (0, 0)),
            pl.BlockSpec((_D * _H,), lambda i: (0,)),
            pl.BlockSpec((_D * _H, _D), lambda i: (0, 0)),
            pl.BlockSpec((_D * _H,), lambda i: (0,)),
            pl.BlockSpec((_D, _D * _H), lambda i: (0, 0)),
            pl.BlockSpec((_D,), lambda i: (0,)),
        ],
        out_specs=pl.BlockSpec((n, _D), lambda i: (0, 0)),
        out_shape=jax.ShapeDtypeStruct((n, _D), jnp.float32),
    )(node_embeddings, node_sign_influence, adj_matrix, adj_matrix,
      Wq, bq, Wk, bk, Wv, bv, Wo, bo)


# speedup vs baseline: 1.9192x; 1.7252x over previous
import math

import jax
import jax.numpy as jnp
from jax import lax
from jax.experimental import pallas as pl
from jax.experimental.pallas import tpu as pltpu

_N = 1024
_D = 64
_H = 2
_NP = 2
_PW = _N // _NP


def _attn_body(xt_ref, sign_ref, adj_ref,
               wqt_ref, bq_ref, wkt_ref, bk_ref, wvt_ref, bv_ref,
               wo_ref, bo_ref, out_ref,
               s_scr, m_scr, vt_scr, wv_scr, den_scr):
    p = pl.program_id(0)
    inv_sqrt_d = 1.0 / math.sqrt(_D)
    bf16 = jnp.bfloat16
    f32 = jnp.float32
    t_lhs = (((0,), (0,)), ((), ()))
    t_rhs = (((1,), (1,)), ((), ()))

    @pl.when(p == 0)
    def _projections_and_scores():
        xt = xt_ref[:].astype(bf16)
        kt = (lax.dot_general(wkt_ref[:].astype(bf16), xt, t_lhs,
                              preferred_element_type=f32)
              + bk_ref[:].reshape(_D * _H, 1)).astype(bf16)
        vt_scr[:] = (lax.dot_general(wvt_ref[:].astype(bf16), xt, t_lhs,
                                     preferred_element_type=f32)
                     + bv_ref[:].reshape(_D * _H, 1)).astype(bf16)
        qt = (lax.dot_general(wqt_ref[:].astype(bf16), xt, t_lhs,
                              preferred_element_type=f32)
              + bq_ref[:].reshape(_D * _H, 1))
        qt = (qt * (sign_ref[:] * inv_sqrt_d)).astype(bf16)
        for h in range(_H):
            qth = qt[h * _D:(h + 1) * _D, :]
            kth = kt[h * _D:(h + 1) * _D, :]
            s = lax.dot_general(qth, kth, t_lhs,
                                preferred_element_type=f32)
            s_scr[h] = s
            m_scr[h] = jnp.max(s, axis=1, keepdims=True)

    mask = adj_ref[:] > 0.0
    for h in range(_H):
        sp = s_scr[h, :, pl.ds(p * _PW, _PW)]
        w = jnp.where(mask, jnp.exp(sp - m_scr[h]), 0.0).astype(bf16)
        den = jnp.sum(w.astype(f32), axis=1, keepdims=True)
        vthp = vt_scr[h * _D:(h + 1) * _D, pl.ds(p * _PW, _PW)]
        wv = lax.dot_general(w, vthp, t_rhs,
                             preferred_element_type=f32)

        @pl.when(p == 0)
        def _init():
            den_scr[h] = den + 1e-10
            wv_scr[h] = wv

        @pl.when(p > 0)
        def _accum():
            den_scr[h] = den_scr[h] + den
            wv_scr[h] = wv_scr[h] + wv

    @pl.when(p == _NP - 1)
    def _finalize():
        heads = jnp.concatenate(
            [wv_scr[h] / den_scr[h] for h in range(_H)], axis=1)
        out_ref[:] = (lax.dot_general(wo_ref[:].astype(bf16),
                                      heads.astype(bf16), t_rhs,
                                      preferred_element_type=f32)
                      + bo_ref[:].reshape(_D, 1))


def kernel(node_embeddings, node_sign_influence, adj_matrix,
           Wq, bq, Wk, bk, Wv, bv, Wo, bo, sign_weight):
    del sign_weight
    n = node_embeddings.shape[0]
    out_t = pl.pallas_call(
        _attn_body,
        grid=(_NP,),
        in_specs=[
            pl.BlockSpec((_D, n), lambda i: (0, 0)),
            pl.BlockSpec((n,), lambda i: (0,)),
            pl.BlockSpec((n, _PW), lambda i: (0, i)),
            pl.BlockSpec((_D, _D * _H), lambda i: (0, 0)),
            pl.BlockSpec((_D * _H,), lambda i: (0,)),
            pl.BlockSpec((_D, _D * _H), lambda i: (0, 0)),
            pl.BlockSpec((_D * _H,), lambda i: (0,)),
            pl.BlockSpec((_D, _D * _H), lambda i: (0, 0)),
            pl.BlockSpec((_D * _H,), lambda i: (0,)),
            pl.BlockSpec((_D, _D * _H), lambda i: (0, 0)),
            pl.BlockSpec((_D,), lambda i: (0,)),
        ],
        out_specs=pl.BlockSpec((_D, n), lambda i: (0, 0)),
        out_shape=jax.ShapeDtypeStruct((_D, n), jnp.float32),
        scratch_shapes=[
            pltpu.VMEM((_H, _N, _N), jnp.float32),
            pltpu.VMEM((_H, _N, 1), jnp.float32),
            pltpu.VMEM((_D * _H, _N), jnp.bfloat16),
            pltpu.VMEM((_H, _N, _D), jnp.float32),
            pltpu.VMEM((_H, _N, 1), jnp.float32),
        ],
        compiler_params=pltpu.CompilerParams(
            dimension_semantics=("arbitrary",),
        ),
    )(node_embeddings.T, node_sign_influence, adj_matrix,
      Wq.T, bq, Wk.T, bk, Wv.T, bv, Wo, bo)
    return out_t.T
